# baseline (speedup 1.0000x reference)
import numpy as np

import jax
import jax.numpy as jnp
from jax.experimental import pallas as pl

_IN_F = 256
_HID = 128
_HEADS = 4
_N = 386
_L = 384
_B = 2
_ALPHA = 0.2
_EPS = 1e-6
_SCALE = float(np.sqrt(_HID * _HEADS))

def _dg(x, y, cx, cy):
    return jax.lax.dot_general(x, y, (((cx,), (cy,)), ((), ())),
                               precision=jax.lax.Precision.DEFAULT,
                               preferred_element_type=jnp.float32)


def _split(x):
    hi = x.astype(jnp.bfloat16)
    lo = (x - hi.astype(jnp.float32)).astype(jnp.bfloat16)
    return hi, lo


def _dg3(x, y, cx, cy):
    xh, xl = _split(x)
    yh, yl = _split(y)
    return (_dg(xh, yh, cx, cy) + _dg(xl, yh, cx, cy)) + _dg(xh, yl, cx, cy)


def _dg2x(x, y, cx, cy):
    xh, xl = _split(x)
    return _dg(xh, y, cx, cy) + _dg(xl, y, cx, cy)


def _gat_body(data_ref, embed_ref, W1s_ref, W1bs_ref, a1blk_ref,
              ln1g_ref, ln1b_ref, W2_ref, W2b_ref, a2_ref, ln2g_ref,
              ln2b_ref, Vw_ref, Vb_ref, out_ref):
    embed = embed_ref[...]
    P = _dg(embed, W1s_ref[...], 1, 1) + W1bs_ref[...]
    m = (data_ref[...] != 0).astype(jnp.float32)
    zc = jnp.zeros((_B, 1), jnp.float32)
    mrow1 = jnp.concatenate([zc, m, zc], axis=1)
    mrow2 = jnp.concatenate([zc, m, zc + 1.0], axis=1)
    mboth = jnp.transpose(mrow1, (1, 0))
    mask2 = jnp.transpose(mrow2, (1, 0))

    S = _dg(P, a1blk_ref[...], 1, 0)
    S2T = jnp.transpose(S[:, _HEADS:], (1, 0))
    outs = ([], [])
    for k in range(_HEADS):
        Pk = P[:, k * _HID:(k + 1) * _HID]
        sc = S[:, k:k + 1] + S2T[k:k + 1, :]
        sc = jnp.where(sc >= 0, sc, _ALPHA * sc)
        T = jnp.exp(sc)
        m0 = mboth[:, 0:1]
        m1 = mboth[:, 1:2]
        rhs = jnp.concatenate([m0 * Pk, m1 * Pk, mboth], axis=1)
        Y = _dg3(T, rhs, 1, 0)
        for b, mb in ((0, m0), (1, m1)):
            num = Y[:, b * _HID:(b + 1) * _HID]
            rs = mb * Y[:, 2 * _HID + b: 2 * _HID + b + 1]
            rs = jnp.where(rs == 0.0, 1.0, rs)
            outs[b].append(mb * num / rs)

    onehot_q = (jax.lax.broadcasted_iota(jnp.int32, (_N, 1), 0)
                == (_N - 1)).astype(jnp.float32)

    res_rows = []
    for b in range(_B):
        h1 = jnp.concatenate(outs[b], axis=1)
        mean = jnp.mean(h1, axis=1, keepdims=True)
        xc = h1 - mean
        var = jnp.sum(xc * xc, axis=1, keepdims=True) * (1.0 / (_HEADS * _HID - 1))
        h1 = ln1g_ref[...] * xc / (jnp.sqrt(var) + _EPS) + ln1b_ref[...]
        h1 = jnp.where(h1 > 0, h1, jnp.exp(jnp.minimum(h1, 0.0)) - 1.0)

        m2 = mask2[:, b:b + 1]
        q_row = _dg(onehot_q, h1, 0, 0)
        uls, urs = [], []
        for k in range(_HEADS):
            W2k = W2_ref[k]
            a2k = a2_ref[k]
            uls.append(_dg(a2k[:, :_HID], W2k, 1, 0))
            urs.append(_dg(a2k[:, _HID:], W2k, 1, 0))
        UR = jnp.concatenate(urs, axis=0)
        SR = _dg(h1, UR, 1, 1)
        wcols = []
        for k in range(_HEADS):
            b2k = W2b_ref[k]
            a2k = a2_ref[k]
            s1q = (jnp.sum(q_row * uls[k])
                   + jnp.sum(b2k * a2k[:, :_HID])
                   + jnp.sum(b2k * a2k[:, _HID:]))
            sc2 = SR[:, k:k + 1] + s1q
            sc2 = jnp.where(sc2 >= 0, sc2, _ALPHA * sc2)
            e = jnp.exp(sc2 * (1.0 / _SCALE)) * m2
            wcols.append(e * (1.0 / jnp.sum(e)))
        W4 = jnp.concatenate(wcols, axis=1)
        V = _dg2x(W4, h1, 0, 0)
        acc = jnp.zeros((1, _HID), jnp.float32)
        for k in range(_HEADS):
            acc = acc + _dg2x(V[k:k + 1, :], W2_ref[k], 1, 1) + W2b_ref[k]
        h2 = acc * (1.0 / _HEADS)
        mean2 = jnp.mean(h2, axis=1, keepdims=True)
        xc2 = h2 - mean2
        var2 = jnp.sum(xc2 * xc2, axis=1, keepdims=True) * (1.0 / (_HID - 1))
        h2 = ln2g_ref[...] * xc2 / (jnp.sqrt(var2) + _EPS) + ln2b_ref[...]
        h2 = jnp.maximum(h2, 0.0)
        res_rows.append(_dg(h2, Vw_ref[...], 1, 1) + Vb_ref[...])

    out_ref[...] = jnp.concatenate(res_rows, axis=0)


def kernel(data, embed1, W1_w, W1_b, a1, ln1_g, ln1_b, W2_w, W2_b, a2,
           ln2_g, ln2_b, V_w, V_b):
    a1s = a1.reshape(_HEADS, 2, _HID) * (1.0 / _SCALE)
    a1blk = jnp.zeros((_HEADS * _HID, 2 * _HEADS), jnp.float32)
    for k in range(_HEADS):
        a1blk = a1blk.at[k * _HID:(k + 1) * _HID, k].set(a1s[k, 0])
        a1blk = a1blk.at[k * _HID:(k + 1) * _HID, _HEADS + k].set(a1s[k, 1])
    return pl.pallas_call(
        _gat_body,
        out_shape=jax.ShapeDtypeStruct((_B, 2), jnp.float32),
    )(data, embed1,
      W1_w.reshape(_HEADS * _HID, _IN_F), W1_b.reshape(1, _HEADS * _HID),
      a1blk, ln1_g.reshape(1, -1), ln1_b.reshape(1, -1),
      W2_w, W2_b.reshape(_HEADS, 1, _HID), a2,
      ln2_g.reshape(1, -1), ln2_b.reshape(1, -1),
      V_w, V_b.reshape(1, 2))


# speedup vs baseline: 2.1231x; 2.1231x over previous
import numpy as np

import jax
import jax.numpy as jnp
from jax.experimental import pallas as pl

_IN_F = 256
_HID = 128
_HEADS = 4
_N = 386
_L = 384
_B = 2
_ALPHA = 0.2
_EPS = 1e-6
_SCALE = float(np.sqrt(_HID * _HEADS))

def _dg(x, y, cx, cy):
    return jax.lax.dot_general(x, y, (((cx,), (cy,)), ((), ())),
                               precision=jax.lax.Precision.DEFAULT,
                               preferred_element_type=jnp.float32)


def _split(x):
    hi = x.astype(jnp.bfloat16)
    lo = (x - hi.astype(jnp.float32)).astype(jnp.bfloat16)
    return hi, lo


def _dg3(x, y, cx, cy):
    xh, xl = _split(x)
    yh, yl = _split(y)
    return (_dg(xh, yh, cx, cy) + _dg(xl, yh, cx, cy)) + _dg(xh, yl, cx, cy)


def _dg2x(x, y, cx, cy):
    xh, xl = _split(x)
    return _dg(xh, y, cx, cy) + _dg(xl, y, cx, cy)


def _gat_body(data_ref, embed_ref, W1s_ref, W1bs_ref, a1_ref,
              ln1g_ref, ln1b_ref, W2_ref, W2b_ref, a2_ref, ln2g_ref,
              ln2b_ref, Vw_ref, Vb_ref, out_ref):
    embed = embed_ref[...]
    P = _dg(embed, W1s_ref[...], 1, 1) + W1bs_ref[...]
    m = (data_ref[...] != 0).astype(jnp.float32)
    zc = jnp.zeros((_B, 1), jnp.float32)
    mrow1 = jnp.concatenate([zc, m, zc], axis=1)
    mrow2 = jnp.concatenate([zc, m, zc + 1.0], axis=1)
    mboth = jnp.transpose(mrow1, (1, 0))
    mask2 = jnp.transpose(mrow2, (1, 0))

    outs = ([], [])
    for k in range(_HEADS):
        Pk = P[:, k * _HID:(k + 1) * _HID]
        a1k = a1_ref[k]
        s1 = _dg(Pk, a1k[:, :_HID], 1, 1) * (1.0 / _SCALE)
        s2 = _dg(a1k[:, _HID:], Pk, 1, 1) * (1.0 / _SCALE)
        sc = s1 + s2
        T = jnp.exp(jnp.maximum(sc, _ALPHA * sc))
        m0 = mboth[:, 0:1]
        m1 = mboth[:, 1:2]
        rhs = jnp.concatenate([m0 * Pk, m1 * Pk, mboth], axis=1)
        Y = _dg3(T, rhs, 1, 0)
        for b, mb in ((0, m0), (1, m1)):
            num = Y[:, b * _HID:(b + 1) * _HID]
            rs = mb * Y[:, 2 * _HID + b: 2 * _HID + b + 1]
            rs = jnp.where(rs == 0.0, 1.0, rs)
            outs[b].append(num * (mb / rs))

    onehot_q = (jax.lax.broadcasted_iota(jnp.int32, (_N, 1), 0)
                == (_N - 1)).astype(jnp.float32)

    res_rows = []
    for b in range(_B):
        h1 = jnp.concatenate(outs[b], axis=1)
        mean = jnp.mean(h1, axis=1, keepdims=True)
        xc = h1 - mean
        var = jnp.sum(xc * xc, axis=1, keepdims=True) * (1.0 / (_HEADS * _HID - 1))
        h1 = ln1g_ref[...] * xc * (1.0 / (jnp.sqrt(var) + _EPS)) + ln1b_ref[...]
        h1 = jnp.where(h1 > 0, h1, jnp.exp(jnp.minimum(h1, 0.0)) - 1.0)

        m2 = mask2[:, b:b + 1]
        q_row = _dg(onehot_q, h1, 0, 0)
        uls, urs = [], []
        for k in range(_HEADS):
            W2k = W2_ref[k]
            a2k = a2_ref[k]
            uls.append(_dg(a2k[:, :_HID], W2k, 1, 0))
            urs.append(_dg(a2k[:, _HID:], W2k, 1, 0))
        UR = jnp.concatenate(urs, axis=0)
        SR = _dg(h1, UR, 1, 1)
        wcols = []
        for k in range(_HEADS):
            b2k = W2b_ref[k]
            a2k = a2_ref[k]
            s1q = (jnp.sum(q_row * uls[k])
                   + jnp.sum(b2k * a2k[:, :_HID])
                   + jnp.sum(b2k * a2k[:, _HID:]))
            sc2 = SR[:, k:k + 1] + s1q
            sc2 = jnp.where(sc2 >= 0, sc2, _ALPHA * sc2)
            e = jnp.exp(sc2 * (1.0 / _SCALE)) * m2
            wcols.append(e * (1.0 / jnp.sum(e)))
        W4 = jnp.concatenate(wcols, axis=1)
        V = _dg2x(W4, h1, 0, 0)
        acc = jnp.zeros((1, _HID), jnp.float32)
        for k in range(_HEADS):
            acc = acc + _dg2x(V[k:k + 1, :], W2_ref[k], 1, 1) + W2b_ref[k]
        h2 = acc * (1.0 / _HEADS)
        mean2 = jnp.mean(h2, axis=1, keepdims=True)
        xc2 = h2 - mean2
        var2 = jnp.sum(xc2 * xc2, axis=1, keepdims=True) * (1.0 / (_HID - 1))
        h2 = ln2g_ref[...] * xc2 / (jnp.sqrt(var2) + _EPS) + ln2b_ref[...]
        h2 = jnp.maximum(h2, 0.0)
        res_rows.append(_dg(h2, Vw_ref[...], 1, 1) + Vb_ref[...])

    out_ref[...] = jnp.concatenate(res_rows, axis=0)


def kernel(data, embed1, W1_w, W1_b, a1, ln1_g, ln1_b, W2_w, W2_b, a2,
           ln2_g, ln2_b, V_w, V_b):
    return pl.pallas_call(
        _gat_body,
        out_shape=jax.ShapeDtypeStruct((_B, 2), jnp.float32),
    )(data, embed1,
      W1_w.reshape(_HEADS * _HID, _IN_F), W1_b.reshape(1, _HEADS * _HID),
      a1, ln1_g.reshape(1, -1), ln1_b.reshape(1, -1),
      W2_w, W2_b.reshape(_HEADS, 1, _HID), a2,
      ln2_g.reshape(1, -1), ln2_b.reshape(1, -1),
      V_w, V_b.reshape(1, 2))


# speedup vs baseline: 2.5008x; 1.1779x over previous
import numpy as np

import jax
import jax.numpy as jnp
from jax.experimental import pallas as pl

_IN_F = 256
_HID = 128
_HEADS = 4
_N = 386
_L = 384
_B = 2
_ALPHA = 0.2
_EPS = 1e-6
_SCALE = float(np.sqrt(_HID * _HEADS))

def _dg(x, y, cx, cy):
    return jax.lax.dot_general(x, y, (((cx,), (cy,)), ((), ())),
                               precision=jax.lax.Precision.DEFAULT,
                               preferred_element_type=jnp.float32)


def _split(x):
    hi = x.astype(jnp.bfloat16)
    lo = (x - hi.astype(jnp.float32)).astype(jnp.bfloat16)
    return hi, lo


def _dg3(x, y, cx, cy):
    xh, xl = _split(x)
    yh, yl = _split(y)
    return (_dg(xh, yh, cx, cy) + _dg(xl, yh, cx, cy)) + _dg(xh, yl, cx, cy)


def _dg2x(x, y, cx, cy):
    xh, xl = _split(x)
    return _dg(xh, y, cx, cy) + _dg(xl, y, cx, cy)


def _gat_body(data_ref, embed_ref, W1s_ref, W1bs_ref, a1_ref,
              ln1g_ref, ln1b_ref, W2_ref, W2b_ref, a2_ref, ln2g_ref,
              ln2b_ref, Vw_ref, Vb_ref, out_ref):
    embed = embed_ref[...]
    P = _dg(embed, W1s_ref[...], 1, 1) + W1bs_ref[...]
    m = (data_ref[...] != 0).astype(jnp.float32)
    zc = jnp.zeros((_B, 1), jnp.float32)
    mrow1 = jnp.concatenate([zc, m, zc], axis=1)
    mrow2 = jnp.concatenate([zc, m, zc + 1.0], axis=1)
    mboth = jnp.transpose(mrow1, (1, 0))

    outs = ([], [])
    for k in range(_HEADS):
        Pk = P[:, k * _HID:(k + 1) * _HID]
        a1k = a1_ref[k]
        s1 = _dg(Pk, a1k[:, :_HID], 1, 1) * (1.0 / _SCALE)
        s2 = _dg(a1k[:, _HID:], Pk, 1, 1) * (1.0 / _SCALE)
        sc = s1 + s2
        T = jnp.exp(jnp.maximum(sc, _ALPHA * sc))
        m0 = mboth[:, 0:1]
        m1 = mboth[:, 1:2]
        rhs = jnp.concatenate([m0 * Pk, m1 * Pk, mboth], axis=1)
        Y = _dg3(T, rhs, 1, 0)
        for b, mb in ((0, m0), (1, m1)):
            num = Y[:, b * _HID:(b + 1) * _HID]
            rs = mb * Y[:, 2 * _HID + b: 2 * _HID + b + 1]
            rs = jnp.where(rs == 0.0, 1.0, rs)
            outs[b].append(num * (mb / rs))

    onehot_q = (jax.lax.broadcasted_iota(jnp.int32, (_N, 1), 0)
                == (_N - 1)).astype(jnp.float32)

    res_rows = []
    for b in range(_B):
        h1 = jnp.concatenate(outs[b], axis=1)
        mean = jnp.mean(h1, axis=1, keepdims=True)
        xc = h1 - mean
        var = jnp.sum(xc * xc, axis=1, keepdims=True) * (1.0 / (_HEADS * _HID - 1))
        h1 = ln1g_ref[...] * xc * (1.0 / (jnp.sqrt(var) + _EPS)) + ln1b_ref[...]
        h1 = jnp.where(h1 > 0, h1, jnp.exp(jnp.minimum(h1, 0.0)) - 1.0)

        m2row = mrow2[b:b + 1, :]
        q_row = _dg(onehot_q, h1, 0, 0)
        uls, urs = [], []
        for k in range(_HEADS):
            W2k = W2_ref[k]
            a2k = a2_ref[k]
            uls.append(_dg(a2k[:, :_HID], W2k, 1, 0))
            urs.append(_dg(a2k[:, _HID:], W2k, 1, 0))
        UR = jnp.concatenate(urs, axis=0)
        SR = _dg(h1, UR, 1, 1)
        SRT = jnp.transpose(SR, (1, 0))
        score_rows = []
        for k in range(_HEADS):
            b2k = W2b_ref[k]
            a2k = a2_ref[k]
            s1q = (jnp.sum(q_row * uls[k])
                   + jnp.sum(b2k * a2k[:, :_HID])
                   + jnp.sum(b2k * a2k[:, _HID:]))
            score_rows.append(SRT[k:k + 1, :] + s1q)
        sc4 = jnp.concatenate(score_rows, axis=0) * (1.0 / _SCALE)
        e4 = jnp.exp(jnp.maximum(sc4, _ALPHA * sc4)) * m2row
        sums = jnp.sum(e4, axis=1, keepdims=True)
        w4 = e4 * (1.0 / sums)
        W4 = jnp.transpose(w4, (1, 0))
        V = _dg2x(W4, h1, 0, 0)
        acc = jnp.zeros((1, _HID), jnp.float32)
        for k in range(_HEADS):
            acc = acc + _dg2x(V[k:k + 1, :], W2_ref[k], 1, 1) + W2b_ref[k]
        h2 = acc * (1.0 / _HEADS)
        mean2 = jnp.mean(h2, axis=1, keepdims=True)
        xc2 = h2 - mean2
        var2 = jnp.sum(xc2 * xc2, axis=1, keepdims=True) * (1.0 / (_HID - 1))
        h2 = ln2g_ref[...] * xc2 / (jnp.sqrt(var2) + _EPS) + ln2b_ref[...]
        h2 = jnp.maximum(h2, 0.0)
        res_rows.append(_dg(h2, Vw_ref[...], 1, 1) + Vb_ref[...])

    out_ref[...] = jnp.concatenate(res_rows, axis=0)


def kernel(data, embed1, W1_w, W1_b, a1, ln1_g, ln1_b, W2_w, W2_b, a2,
           ln2_g, ln2_b, V_w, V_b):
    return pl.pallas_call(
        _gat_body,
        out_shape=jax.ShapeDtypeStruct((_B, 2), jnp.float32),
    )(data, embed1,
      W1_w.reshape(_HEADS * _HID, _IN_F), W1_b.reshape(1, _HEADS * _HID),
      a1, ln1_g.reshape(1, -1), ln1_b.reshape(1, -1),
      W2_w, W2_b.reshape(_HEADS, 1, _HID), a2,
      ln2_g.reshape(1, -1), ln2_b.reshape(1, -1),
      V_w, V_b.reshape(1, 2))
